# baseline (speedup 1.0000x reference)
import functools

import jax
import jax.numpy as jnp
from jax import lax
from jax.experimental import pallas as pl
from jax.experimental.pallas import tpu as pltpu
from jax.experimental.pallas import tpu_sc as plsc

_B = 64
_N = 100000
_K = 3
_J = 64


def _make_sc_gather():
    info = plsc.get_sparse_core_info()
    nc, ns = info.num_cores, info.num_subcores
    nw = nc * ns
    bpw = _B // nw

    mesh = plsc.VectorSubcoreMesh(core_axis_name="c", subcore_axis_name="s")

    @functools.partial(
        pl.kernel,
        mesh=mesh,
        out_type=jax.ShapeDtypeStruct((_B, _J, _K), jnp.float32),
        scratch_types=[
            pltpu.VMEM((_J,), jnp.int32),
            pltpu.VMEM((bpw, _J, _K), jnp.float32),
            pltpu.SemaphoreType.DMA,
        ],
    )
    def sc_gather(pos_hbm, idx_hbm, out_hbm, idx_vm, rows_v, sem):
        nl = 16
        wid = lax.axis_index("s") * nc + lax.axis_index("c")
        pltpu.sync_copy(idx_hbm, idx_vm)
        idx_vecs = [idx_vm[pl.ds(g * nl, nl)] for g in range(_J // nl)]
        for t in range(bpw):
            b = wid * bpw + t
            for j in range(_J):
                row = idx_vecs[j // nl][j % nl]
                pltpu.make_async_copy(
                    pos_hbm.at[b, row], rows_v.at[t, j], sem
                ).start()
        for t in range(bpw):
            b = wid * bpw + t
            for j in range(_J):
                pltpu.make_async_copy(
                    pos_hbm.at[b, 0], rows_v.at[t, j], sem
                ).wait()
            pltpu.sync_copy(rows_v.at[t], out_hbm.at[b])

    return sc_gather


_sc_gather = _make_sc_gather()


@jax.jit
def kernel(pos, idx):
    idx32 = idx.astype(jnp.int32)
    out = _sc_gather(pos, idx32)
    return out.reshape(_B, _J * _K)


# speedup vs baseline: 3.7146x; 3.7146x over previous
import functools

import jax
import jax.numpy as jnp
from jax import lax
from jax.experimental import pallas as pl
from jax.experimental.pallas import tpu as pltpu
from jax.experimental.pallas import tpu_sc as plsc

_B = 64
_N = 100000
_K = 3
_J = 64
_T = 128


def _make_sc_gather():
    info = plsc.get_sparse_core_info()
    nc, ns, nl = info.num_cores, info.num_subcores, info.num_lanes
    nw = nc * ns
    jpw = _J // nw
    ng = (_B * _K) // nl

    mesh = plsc.VectorSubcoreMesh(core_axis_name="c", subcore_axis_name="s")

    @functools.partial(
        pl.kernel,
        mesh=mesh,
        compiler_params=pltpu.CompilerParams(needs_layout_passes=False),
        out_type=jax.ShapeDtypeStruct((nw, _B, jpw * _K), jnp.float32),
        scratch_types=[
            pltpu.VMEM((_J,), jnp.int32),
            pltpu.VMEM((jpw, _B, 2 * _T), jnp.float32),
            pltpu.VMEM((_B, jpw * _K), jnp.float32),
            pltpu.SemaphoreType.DMA,
        ],
    )
    def sc_gather(pos_hbm, idx_hbm, out_hbm, idx_vm, win_v, slab_v, sem):
        wid = lax.axis_index("s") * nc + lax.axis_index("c")
        pltpu.sync_copy(idx_hbm, idx_vm)
        offs = []
        copies = []
        for t in range(jpw):
            j = wid * jpw + t
            lanes = jnp.full((nl,), j, dtype=jnp.int32)
            idx_j = jnp.max(plsc.load_gather(idx_vm, [lanes]))
            c = idx_j * _K
            c0 = pl.multiple_of((c // _T) * _T, _T)
            c1 = pl.multiple_of(((c + _K - 1) // _T) * _T, _T)
            offs.append(c - c0)
            cp0 = pltpu.make_async_copy(
                pos_hbm.at[:, pl.ds(c0, _T)], win_v.at[t, :, pl.ds(0, _T)], sem
            )
            cp1 = pltpu.make_async_copy(
                pos_hbm.at[:, pl.ds(c1, _T)], win_v.at[t, :, pl.ds(_T, _T)], sem
            )
            cp0.start()
            cp1.start()
            copies += [cp0, cp1]
        for cp in copies:
            cp.wait()
        for t in range(jpw):
            d = offs[t]
            for g in range(ng):
                e = lax.iota(jnp.int32, 16) + g * nl
                b_vec = e // _K
                u_vec = e % _K
                vals = plsc.load_gather(win_v.at[t], [b_vec, d + u_vec])
                plsc.store_scatter(slab_v, [b_vec, u_vec + t * _K], vals)
        pltpu.sync_copy(slab_v, out_hbm.at[wid])

    return sc_gather


_sc_gather = _make_sc_gather()


@jax.jit
def kernel(pos, idx):
    pos2d = pos.reshape(_B, _N * _K)
    idx32 = idx.astype(jnp.int32)
    out3 = _sc_gather(pos2d, idx32)
    return out3.transpose(1, 0, 2).reshape(_B, _J * _K)


# speedup vs baseline: 56.6678x; 15.2554x over previous
import functools

import jax
import jax.numpy as jnp
from jax import lax
from jax.experimental import pallas as pl
from jax.experimental.pallas import tpu as pltpu
from jax.experimental.pallas import tpu_sc as plsc

_B = 64
_N = 100000
_K = 3
_J = 64
_T = 128


def _make_sc_gather():
    info = plsc.get_sparse_core_info()
    nc, ns, nl = info.num_cores, info.num_subcores, info.num_lanes
    nw = nc * ns
    jpw = _J // nw
    ng = (_B * _K) // nl

    mesh = plsc.VectorSubcoreMesh(core_axis_name="c", subcore_axis_name="s")

    @functools.partial(
        pl.kernel,
        mesh=mesh,
        compiler_params=pltpu.CompilerParams(needs_layout_passes=False),
        out_type=jax.ShapeDtypeStruct((nw, _B, jpw * _K), jnp.float32),
        scratch_types=[
            pltpu.VMEM((_J,), jnp.int32),
            pltpu.VMEM((jpw, _K, _B, _T), jnp.float32),
            pltpu.VMEM((_B, jpw * _K), jnp.float32),
            pltpu.SemaphoreType.DMA,
        ],
    )
    def sc_gather(pos_hbm, idx_hbm, out_hbm, idx_vm, win_v, slab_v, sem):
        wid = lax.axis_index("s") * nc + lax.axis_index("c")
        pltpu.sync_copy(idx_hbm, idx_vm)
        offs = []
        copies = []
        for t in range(jpw):
            j = wid * jpw + t
            lanes = jnp.full((nl,), j, dtype=jnp.int32)
            idx_j = jnp.max(plsc.load_gather(idx_vm, [lanes]))
            c0 = pl.multiple_of((idx_j // _T) * _T, _T)
            offs.append(idx_j - c0)
            cp = pltpu.make_async_copy(
                pos_hbm.at[:, :, pl.ds(c0, _T)], win_v.at[t], sem
            )
            cp.start()
            copies.append(cp)
        for cp in copies:
            cp.wait()
        for t in range(jpw):
            d = offs[t]
            for g in range(ng):
                e = lax.iota(jnp.int32, nl) + g * nl
                b_vec = e // _K
                k_vec = e % _K
                vals = plsc.load_gather(
                    win_v.at[t], [k_vec, b_vec, jnp.full((nl,), d, jnp.int32)]
                )
                plsc.store_scatter(slab_v, [b_vec, k_vec + t * _K], vals)
        pltpu.sync_copy(slab_v, out_hbm.at[wid])

    return sc_gather


_sc_gather = _make_sc_gather()


@jax.jit
def kernel(pos, idx):
    pos_t = jnp.transpose(pos, (2, 0, 1))
    idx32 = idx.astype(jnp.int32)
    out3 = _sc_gather(pos_t, idx32)
    return out3.transpose(1, 0, 2).reshape(_B, _J * _K)


# speedup vs baseline: 58.0894x; 1.0251x over previous
import functools

import jax
import jax.numpy as jnp
from jax import lax
from jax.experimental import pallas as pl
from jax.experimental.pallas import tpu as pltpu
from jax.experimental.pallas import tpu_sc as plsc

_B = 64
_N = 100000
_K = 3
_J = 64
_T = 128


def _make_sc_gather():
    info = plsc.get_sparse_core_info()
    nc, ns, nl = info.num_cores, info.num_subcores, info.num_lanes
    bps = _B // nc
    jpw = _J // ns
    ng = (bps * _K) // nl

    mesh = plsc.VectorSubcoreMesh(core_axis_name="c", subcore_axis_name="s")

    @functools.partial(
        pl.kernel,
        mesh=mesh,
        compiler_params=pltpu.CompilerParams(needs_layout_passes=False),
        out_type=jax.ShapeDtypeStruct((_B, ns, jpw * _K), jnp.float32),
        scratch_types=[
            pltpu.VMEM((_J,), jnp.int32),
            pltpu.VMEM((jpw, _K, bps, _T), jnp.float32),
            pltpu.VMEM((bps, 1, jpw * _K), jnp.float32),
            pltpu.SemaphoreType.DMA,
        ],
    )
    def sc_gather(pos_hbm, idx_hbm, out_hbm, idx_vm, win_v, slab_v, sem):
        sc = lax.axis_index("c")
        s = lax.axis_index("s")
        b0 = sc * bps
        pltpu.sync_copy(idx_hbm, idx_vm)
        offs = []
        copies = []
        for t in range(jpw):
            j = s * jpw + t
            lanes = jnp.full((nl,), j, dtype=jnp.int32)
            idx_j = jnp.max(plsc.load_gather(idx_vm, [lanes]))
            c0 = pl.multiple_of((idx_j // _T) * _T, _T)
            offs.append(idx_j - c0)
            cp = pltpu.make_async_copy(
                pos_hbm.at[:, pl.ds(b0, bps), pl.ds(c0, _T)], win_v.at[t], sem
            )
            cp.start()
            copies.append(cp)
        for cp in copies:
            cp.wait()
        for t in range(jpw):
            d = offs[t]
            for g in range(ng):
                e = lax.iota(jnp.int32, nl) + g * nl
                b_vec = e // _K
                k_vec = e % _K
                vals = plsc.load_gather(
                    win_v.at[t], [k_vec, b_vec, jnp.full((nl,), d, jnp.int32)]
                )
                plsc.store_scatter(
                    slab_v, [b_vec, jnp.zeros((nl,), jnp.int32), k_vec + t * _K], vals
                )
        pltpu.sync_copy(slab_v, out_hbm.at[pl.ds(b0, bps), pl.ds(s, 1), :])

    return sc_gather


_sc_gather = _make_sc_gather()


@jax.jit
def kernel(pos, idx):
    pos_t = jnp.transpose(pos, (2, 0, 1))
    idx32 = idx.astype(jnp.int32)
    out3 = _sc_gather(pos_t, idx32)
    return out3.reshape(_B, _J * _K)


# speedup vs baseline: 58.2247x; 1.0023x over previous
import functools

import jax
import jax.numpy as jnp
from jax import lax
from jax.experimental import pallas as pl
from jax.experimental.pallas import tpu as pltpu
from jax.experimental.pallas import tpu_sc as plsc

_B = 64
_N = 100000
_K = 3
_J = 64
_T = 128


def _make_sc_gather():
    info = plsc.get_sparse_core_info()
    nc, ns, nl = info.num_cores, info.num_subcores, info.num_lanes
    bps = _B // nc
    jpw = _J // ns
    ng = (bps * _K) // nl

    mesh = plsc.VectorSubcoreMesh(core_axis_name="c", subcore_axis_name="s")

    @functools.partial(
        pl.kernel,
        mesh=mesh,
        compiler_params=pltpu.CompilerParams(
            needs_layout_passes=False, skip_device_barrier=True
        ),
        out_type=jax.ShapeDtypeStruct((_B, ns, jpw * _K), jnp.float32),
        scratch_types=[
            pltpu.VMEM((_J,), jnp.int32),
            pltpu.VMEM((jpw, _K, bps, _T), jnp.float32),
            pltpu.VMEM((bps, 1, jpw * _K), jnp.float32),
            pltpu.SemaphoreType.DMA,
        ],
    )
    def sc_gather(pos_hbm, idx_hbm, out_hbm, idx_vm, win_v, slab_v, sem):
        sc = lax.axis_index("c")
        s = lax.axis_index("s")
        b0 = sc * bps
        pltpu.sync_copy(idx_hbm, idx_vm)
        offs = []
        copies = []
        for t in range(jpw):
            j = s * jpw + t
            lanes = jnp.full((nl,), j, dtype=jnp.int32)
            idx_j = jnp.max(plsc.load_gather(idx_vm, [lanes]))
            c0 = pl.multiple_of((idx_j // _T) * _T, _T)
            offs.append(idx_j - c0)
            cp = pltpu.make_async_copy(
                pos_hbm.at[:, pl.ds(b0, bps), pl.ds(c0, _T)], win_v.at[t], sem
            )
            cp.start()
            copies.append(cp)
        for cp in copies:
            cp.wait()
        for t in range(jpw):
            d = offs[t]
            for g in range(ng):
                e = lax.iota(jnp.int32, nl) + g * nl
                b_vec = e // _K
                k_vec = e % _K
                vals = plsc.load_gather(
                    win_v.at[t], [k_vec, b_vec, jnp.full((nl,), d, jnp.int32)]
                )
                plsc.store_scatter(
                    slab_v, [b_vec, jnp.zeros((nl,), jnp.int32), k_vec + t * _K], vals
                )
        pltpu.sync_copy(slab_v, out_hbm.at[pl.ds(b0, bps), pl.ds(s, 1), :])

    return sc_gather


_sc_gather = _make_sc_gather()


@jax.jit
def kernel(pos, idx):
    pos_t = jnp.transpose(pos, (2, 0, 1))
    idx32 = idx.astype(jnp.int32)
    out3 = _sc_gather(pos_t, idx32)
    return out3.reshape(_B, _J * _K)
